# baseline (speedup 1.0000x reference)
import functools

import jax
import jax.numpy as jnp
from jax import lax
from jax.experimental import pallas as pl
from jax.experimental.pallas import tpu as pltpu

_TOP_K = 50
_MIN_KEEP = 5
_RB = 8
_CH = 2048
_NEG = -3e38
_PAD_VAL = -1e30


def _body(m_ref, ids_ref, topp_ref, temp_ref, pen_ref, probs_ref, tok_ref):
    vpad = m_ref.shape[1]
    nch = vpad // _CH
    L = ids_ref.shape[1]
    pen = pen_ref[0, 0]
    topp = topp_ref[0, 0]
    temp = temp_ref[0, 0]

    idsv = ids_ref[...]
    lane_l = lax.broadcasted_iota(jnp.int32, (_RB, L), 1)
    lane64 = lax.broadcasted_iota(jnp.int32, (_RB, 64), 1)

    def chunk_body(c, cmax):
        off = pl.multiple_of(c * _CH, _CH)
        xc = m_ref[:, pl.ds(off, _CH)]
        posc = c * _CH + lax.broadcasted_iota(jnp.int32, (_RB, _CH), 1)

        def l_body(l, mk):
            sel = (lane_l == l)
            idl = jnp.sum(jnp.where(sel, idsv, 0), axis=1, keepdims=True)
            return mk | (posc == idl).astype(jnp.int32)

        mk = lax.fori_loop(0, L, l_body, jnp.zeros((_RB, _CH), jnp.int32))
        xpen = jnp.where(xc < 0, xc * pen, xc / pen)
        xc2 = jnp.where(mk != 0, xpen, xc)
        m_ref[:, pl.ds(off, _CH)] = xc2
        cmx = jnp.max(xc2, axis=1, keepdims=True)
        return jnp.where(lane64 == c, cmx, cmax)

    lax.fori_loop(0, nch, chunk_body, jnp.full((_RB, 64), _NEG, jnp.float32))

    pos_full = lax.broadcasted_iota(jnp.int32, (_RB, vpad), 1)

    def extract_body(k, carry):
        vals, toks = carry
        x = m_ref[...]
        vmax = jnp.max(x, axis=1, keepdims=True)
        cand = jnp.where(x == vmax, pos_full, jnp.int32(vpad))
        idx = jnp.min(cand, axis=1, keepdims=True)
        m_ref[...] = jnp.where(pos_full == idx, _NEG, x)
        vals = jnp.where(lane64 == k, vmax, vals)
        toks = jnp.where(lane64 == k, idx, toks)
        return vals, toks

    vals, toks = lax.fori_loop(
        0, _TOP_K, extract_body,
        (jnp.full((_RB, 64), _NEG, jnp.float32),
         jnp.zeros((_RB, 64), jnp.int32)))

    valid = lane64 < _TOP_K
    logits = vals / temp
    lmax = jnp.max(jnp.where(valid, logits, _NEG), axis=1, keepdims=True)
    e = jnp.where(valid, jnp.exp(logits - lmax), 0.0)
    p0 = e / jnp.sum(e, axis=1, keepdims=True)
    r64 = lax.broadcasted_iota(jnp.int32, (64, 64), 0)
    c64 = lax.broadcasted_iota(jnp.int32, (64, 64), 1)
    tri = (r64 <= c64).astype(jnp.float32)
    cum = jnp.dot(p0, tri, preferred_element_type=jnp.float32)
    mask = (cum < topp) | (lane64 < _MIN_KEEP)
    filt = jnp.where(mask & valid, logits, jnp.float32(-1000.0))
    fmax = jnp.max(jnp.where(valid, filt, _NEG), axis=1, keepdims=True)
    e2 = jnp.where(valid, jnp.exp(filt - fmax), 0.0)
    probs = e2 / jnp.sum(e2, axis=1, keepdims=True)

    probs_ref[...] = probs[:, :_TOP_K]
    tok_ref[...] = toks[:, :_TOP_K]


@jax.jit
def kernel(m_logits, input_ids, top_p, temperature, penalty):
    B, V = m_logits.shape
    vpad = ((V + _CH - 1) // _CH) * _CH
    mp = jnp.pad(m_logits, ((0, 0), (0, vpad - V)), constant_values=_PAD_VAL)
    grid = (B // _RB,)
    probs, tok = pl.pallas_call(
        _body,
        grid=grid,
        in_specs=[
            pl.BlockSpec((_RB, vpad), lambda i: (i, 0)),
            pl.BlockSpec((_RB, input_ids.shape[1]), lambda i: (i, 0)),
            pl.BlockSpec(memory_space=pltpu.SMEM),
            pl.BlockSpec(memory_space=pltpu.SMEM),
            pl.BlockSpec(memory_space=pltpu.SMEM),
        ],
        out_specs=[
            pl.BlockSpec((_RB, _TOP_K), lambda i: (i, 0)),
            pl.BlockSpec((_RB, _TOP_K), lambda i: (i, 0)),
        ],
        out_shape=[
            jax.ShapeDtypeStruct((B, _TOP_K), jnp.float32),
            jax.ShapeDtypeStruct((B, _TOP_K), jnp.int32),
        ],
    )(mp, input_ids, top_p, temperature, penalty)
    return (probs, tok)


# speedup vs baseline: 21.0913x; 21.0913x over previous
import functools

import jax
import jax.numpy as jnp
from jax import lax
from jax.experimental import pallas as pl
from jax.experimental.pallas import tpu as pltpu
from jax.experimental.pallas import tpu_sc as plsc

_TOP_K = 50
_MIN_KEEP = 5
_RB = 8
_CH = 1024
_POS_BIG = 3e38
_NEG = -3e38
_PAD_VAL = -1e30
_NC = 2
_NS = 16
_LIDS = 256


def _sc_penalty_body(mpf_hbm, ids_hbm, pen_hbm, tok_hbm,
                     ids_v, idx_v, vals_v, pen_v, sem):
    vpad = mpf_hbm.shape[0] // ids_hbm.shape[0]
    rows_per_w = ids_hbm.shape[0] // (_NC * _NS)
    wid = lax.axis_index("s") * _NC + lax.axis_index("c")
    pltpu.sync_copy(pen_hbm, pen_v)
    pv = pen_v[...]
    nh = _LIDS // 128
    for j in range(rows_per_w):
        r = wid * rows_per_w + j
        pltpu.sync_copy(ids_hbm.at[r], ids_v)
        for h in range(nh):
            for k in range(128 // 16):
                iv = ids_v[h, pl.ds(k * 16, 16)]
                idx_v[h, pl.ds(k * 16, 16)] = iv + r * vpad
        gathers = [
            pltpu.async_copy(mpf_hbm.at[idx_v.at[h]], vals_v.at[h], sem)
            for h in range(nh)]
        for g in gathers:
            g.wait()
        for h in range(nh):
            for k in range(128 // 16):
                x = vals_v[h, pl.ds(k * 16, 16)]
                vals_v[h, pl.ds(k * 16, 16)] = jnp.where(x < 0, x * pv, x / pv)
        scatters = [
            pltpu.async_copy(vals_v.at[h], mpf_hbm.at[idx_v.at[h]], sem)
            for h in range(nh)]
        for s in scatters:
            s.wait()
    pltpu.sync_copy(pen_v, tok_hbm.at[wid])


def _apply_penalty_sc(mpf, B, vpad, input_ids, penalty):
    L = input_ids.shape[1]
    ids_p = jnp.pad(input_ids, ((0, 0), (0, _LIDS - L)), mode="edge")
    ids_p = ids_p.reshape(B, _LIDS // 128, 128)
    pen16 = jnp.broadcast_to(penalty.reshape(1), (16,))
    mesh = plsc.VectorSubcoreMesh(core_axis_name="c", subcore_axis_name="s")
    fn = functools.partial(
        pl.kernel,
        mesh=mesh,
        out_type=jax.ShapeDtypeStruct((_NC * _NS, 16), jnp.float32),
        scratch_types=[
            pltpu.VMEM((_LIDS // 128, 128), jnp.int32),
            pltpu.VMEM((_LIDS // 128, 128), jnp.int32),
            pltpu.VMEM((_LIDS // 128, 128), jnp.float32),
            pltpu.VMEM((16,), jnp.float32),
            pltpu.SemaphoreType.DMA,
        ],
        compiler_params=pltpu.CompilerParams(has_side_effects=True),
    )(_sc_penalty_body)
    token = fn(mpf, ids_p, pen16)
    mpf_dep, _ = lax.optimization_barrier((mpf, token))
    return mpf_dep.reshape(B, vpad)


def _tc_body(m_ref, vals_ref, tok_ref):
    vpad = m_ref.shape[1]
    nch = vpad // _CH
    nchp = 128

    lane_nch = lax.broadcasted_iota(jnp.int32, (_RB, nchp), 1)
    lane_nch1 = lax.broadcasted_iota(jnp.int32, (1, nchp), 1)
    lane64 = lax.broadcasted_iota(jnp.int32, (1, 64), 1)
    iotc = lax.broadcasted_iota(jnp.int32, (1, _CH), 1)

    cm = jnp.full((_RB, nchp), _NEG, jnp.float32)
    for c in range(nch):
        xc = m_ref[:, c * _CH:(c + 1) * _CH]
        cmx = jnp.max(xc, axis=1, keepdims=True)
        cm = jnp.where(lane_nch == c, cmx, cm)

    def extract_body(k, carry):
        vals_l, toks_l, cm_l, vlast_l, ilast_l = carry
        new = [[], [], [], [], []]
        for r in range(_RB):
            cm_r = cm_l[r]
            vlast = vlast_l[r]
            ilast = ilast_l[r]
            vmax = jnp.max(cm_r, axis=1, keepdims=True)
            c_idx = jnp.min(jnp.where(cm_r == vmax, lane_nch1, nch))
            off = pl.multiple_of(c_idx * _CH, _CH)
            chunk = m_ref[pl.ds(r, 1), pl.ds(off, _CH)]
            iotg = iotc + off
            hit = (chunk == vmax) & ((vmax < vlast) | (iotg > ilast))
            lidx = jnp.min(jnp.where(hit, iotg, jnp.int32(2**31 - 1)),
                           axis=1, keepdims=True)
            elig = (chunk < vmax) | ((chunk == vmax) & (iotg > lidx))
            ncm = jnp.max(jnp.where(elig, chunk, _NEG),
                          axis=1, keepdims=True)
            new[0].append(jnp.where(lane64 == k, vmax, vals_l[r]))
            new[1].append(jnp.where(lane64 == k, lidx, toks_l[r]))
            new[2].append(jnp.where(lane_nch1 == c_idx, ncm, cm_r))
            new[3].append(vmax)
            new[4].append(lidx)
        return tuple(tuple(x) for x in new)

    init = (tuple(jnp.full((1, 64), _NEG, jnp.float32) for _ in range(_RB)),
            tuple(jnp.zeros((1, 64), jnp.int32) for _ in range(_RB)),
            tuple(cm[r, :][None, :] for r in range(_RB)),
            tuple(jnp.full((1, 1), _POS_BIG, jnp.float32)
                  for _ in range(_RB)),
            tuple(jnp.full((1, 1), -1, jnp.int32) for _ in range(_RB)))
    vals_l, toks_l, _, _, _ = lax.fori_loop(0, _TOP_K, extract_body, init)
    vals = jnp.concatenate(list(vals_l), axis=0)
    toks = jnp.concatenate(list(toks_l), axis=0)

    vals_ref[...] = vals[:, :_TOP_K]
    tok_ref[...] = toks[:, :_TOP_K]


@jax.jit
def kernel(m_logits, input_ids, top_p, temperature, penalty):
    B, V = m_logits.shape
    vpad = ((V + _CH - 1) // _CH) * _CH
    mp = jnp.pad(m_logits, ((0, 0), (0, vpad - V)), constant_values=_PAD_VAL)
    mp = _apply_penalty_sc(mp.reshape(-1), B, vpad, input_ids, penalty)
    vals, tok = pl.pallas_call(
        _tc_body,
        grid=(B // _RB,),
        in_specs=[
            pl.BlockSpec((_RB, vpad), lambda i: (i, 0)),
        ],
        out_specs=[
            pl.BlockSpec((_RB, _TOP_K), lambda i: (i, 0)),
            pl.BlockSpec((_RB, _TOP_K), lambda i: (i, 0)),
        ],
        out_shape=[
            jax.ShapeDtypeStruct((B, _TOP_K), jnp.float32),
            jax.ShapeDtypeStruct((B, _TOP_K), jnp.int32),
        ],
    )(mp)
    logits = vals / temperature
    cumulative_probs = jnp.cumsum(jax.nn.softmax(logits, axis=1), axis=1)
    mask = cumulative_probs < top_p
    keep = jnp.zeros((1, _TOP_K), dtype=bool).at[0, :_MIN_KEEP].set(True)
    mask = mask | keep
    filtered_logits = jnp.where(mask, logits, jnp.float32(-1000.0))
    probs = jax.nn.softmax(filtered_logits, axis=1)
    return (probs, tok)
